# baseline (speedup 1.0000x reference)
import functools

import jax
import jax.numpy as jnp
from jax import lax
from jax.experimental import pallas as pl
from jax.experimental.pallas import tpu as pltpu
from jax.experimental.pallas import tpu_sc as plsc

_B = 4096
_D = 64
_ROW_BLK = 256
_L = 16


def _sc_gather(targets, emb_table, bias16):
    info = plsc.get_sparse_core_info()
    nc, ns = info.num_cores, info.num_subcores
    nw = nc * ns
    bpw = _B // nw

    mesh = plsc.VectorSubcoreMesh(core_axis_name="c", subcore_axis_name="s")

    @functools.partial(
        pl.kernel,
        mesh=mesh,
        compiler_params=pltpu.CompilerParams(
            use_tc_tiling_on_sc=False, needs_layout_passes=False),
        out_type=[
            jax.ShapeDtypeStruct((_B, _D), jnp.float32),
            jax.ShapeDtypeStruct((_B,), jnp.float32),
        ],
        scratch_types=[
            pltpu.VMEM((bpw,), jnp.int32),
            pltpu.VMEM((bpw,), jnp.int32),
            pltpu.VMEM((bpw, _D), jnp.float32),
            pltpu.VMEM((bpw, _L), jnp.float32),
            pltpu.VMEM((bpw,), jnp.float32),
            pltpu.SemaphoreType.DMA,
            pltpu.SemaphoreType.DMA,
        ],
    )
    def gather_kernel(tgt_hbm, emb_hbm, bias16_hbm, rows_out, bias_out,
                      idx_v, idx16_v, rows_v, b16_v, bias_v, sem_e, sem_b):
        wid = lax.axis_index("s") * nc + lax.axis_index("c")
        base = wid * bpw
        pltpu.sync_copy(tgt_hbm.at[pl.ds(base, bpw)], idx_v)
        for k in range(bpw // _L):
            sl = pl.ds(k * _L, _L)
            idx16_v[sl] = lax.shift_right_logical(idx_v[sl], 4)
        ce = pltpu.async_copy(emb_hbm.at[idx_v], rows_v, sem_e)
        cb = pltpu.async_copy(bias16_hbm.at[idx16_v], b16_v, sem_b)
        ce.wait()
        cb.wait()
        for k in range(bpw // _L):
            sl = pl.ds(k * _L, _L)
            row_ids = lax.iota(jnp.int32, _L) + (k * _L)
            col_ids = lax.bitwise_and(idx_v[sl], _L - 1)
            bias_v[sl] = plsc.load_gather(b16_v, [row_ids, col_ids])
        pltpu.sync_copy(rows_v, rows_out.at[pl.ds(base, bpw)])
        pltpu.sync_copy(bias_v, bias_out.at[pl.ds(base, bpw)])

    return gather_kernel(targets, emb_table, bias16)


def _broadcast_body(u_ref, g_ref, b_ref, out_ref, dot_scr):
    @pl.when(pl.program_id(0) == 0)
    def _():
        dot_scr[...] = jnp.sum(u_ref[...] * g_ref[...], axis=1)[None, :]

    out_ref[...] = b_ref[...] + dot_scr[...]


def kernel(user_representations, targets, emb_table, bias_table):
    targets = targets.astype(jnp.int32)
    bias16 = bias_table.reshape(bias_table.shape[0] // _L, _L)
    g, bias_g = _sc_gather(targets, emb_table, bias16)
    return pl.pallas_call(
        _broadcast_body,
        grid=(_B // _ROW_BLK,),
        in_specs=[
            pl.BlockSpec((_B, _D), lambda i: (0, 0)),
            pl.BlockSpec((_B, _D), lambda i: (0, 0)),
            pl.BlockSpec((_ROW_BLK, 1), lambda i: (i, 0)),
        ],
        out_specs=pl.BlockSpec((_ROW_BLK, _B), lambda i: (i, 0)),
        out_shape=jax.ShapeDtypeStruct((_B, _B), jnp.float32),
        scratch_shapes=[pltpu.VMEM((1, _B), jnp.float32)],
    )(user_representations, g, bias_g.reshape(_B, 1))
